# Initial kernel scaffold; baseline (speedup 1.0000x reference)
import jax
import jax.numpy as jnp
from jax.experimental import pallas as pl


def kernel(text, W_word, W_ctx):
    raise NotImplementedError("write your pallas kernel here")


# speedup vs baseline: 1.0861x; 1.0861x over previous
import functools
import jax
import jax.numpy as jnp
from jax import lax
from jax.experimental import pallas as pl
from jax.experimental.pallas import tpu as pltpu
from jax.experimental.pallas import tpu_sc as plsc

VOCAB = 1000000
DIM = 32
BATCH = 16384
HIST = 50

NC = 2
NS = 16
NW = NC * NS

N = BATCH * HIST
IDX_CHUNK = 128
ROWS = N // IDX_CHUNK
ROWS_PER_W = ROWS // NW


def _make_gather():
    mesh = plsc.VectorSubcoreMesh(core_axis_name="c", subcore_axis_name="s")

    @functools.partial(
        pl.kernel,
        mesh=mesh,
        out_type=[
            jax.ShapeDtypeStruct((N, DIM), jnp.float32),
            jax.ShapeDtypeStruct((N, DIM), jnp.float32),
        ],
        scratch_types=[
            pltpu.VMEM((IDX_CHUNK,), jnp.int32),
            pltpu.VMEM((IDX_CHUNK, DIM), jnp.float32),
            pltpu.VMEM((IDX_CHUNK, DIM), jnp.float32),
            pltpu.SemaphoreType.DMA,
            pltpu.SemaphoreType.DMA,
        ],
        compiler_params=pltpu.CompilerParams(use_tc_tiling_on_sc=False),
    )
    def gather2(idx_hbm, word_hbm, ctx_hbm, out_w, out_c,
                idx_v, rows_w, rows_c, sem_w, sem_c):
        wid = lax.axis_index("s") * NC + lax.axis_index("c")
        row0 = wid * ROWS_PER_W

        def body(g, carry):
            row = row0 + g
            pltpu.sync_copy(idx_hbm.at[row], idx_v)
            cw = pltpu.async_copy(word_hbm.at[idx_v], rows_w, sem_w)
            cc = pltpu.async_copy(ctx_hbm.at[idx_v], rows_c, sem_c)
            cw.wait()
            cc.wait()
            base = row * IDX_CHUNK
            pltpu.sync_copy(rows_w, out_w.at[pl.ds(base, IDX_CHUNK)])
            pltpu.sync_copy(rows_c, out_c.at[pl.ds(base, IDX_CHUNK)])
            return carry

        lax.fori_loop(0, ROWS_PER_W, body, 0)

    return gather2


_gather2 = _make_gather()


@jax.jit
def kernel(text, W_word, W_ctx):
    idx = text.reshape(ROWS, IDX_CHUNK).astype(jnp.int32)
    out_w, out_c = _gather2(idx, W_word, W_ctx)
    contextMatrix = out_c.reshape(BATCH, HIST, DIM)
    wordMatrix = out_w.reshape(BATCH, HIST, DIM)
    return (contextMatrix, wordMatrix)


# speedup vs baseline: 1.1751x; 1.0820x over previous
import functools
import jax
import jax.numpy as jnp
from jax import lax
from jax.experimental import pallas as pl
from jax.experimental.pallas import tpu as pltpu
from jax.experimental.pallas import tpu_sc as plsc

VOCAB = 1000000
DIM = 32
BATCH = 16384
HIST = 50

NC = 2
NS = 16
NW = NC * NS

N = BATCH * HIST
IDX_CHUNK = 128
ROWS = N // IDX_CHUNK
ROWS_PER_W = ROWS // NW
K = 4
GROUPS = ROWS_PER_W // K
GROUP_ROWS = K * IDX_CHUNK


def _make_gather():
    mesh = plsc.VectorSubcoreMesh(core_axis_name="c", subcore_axis_name="s")

    @functools.partial(
        pl.kernel,
        mesh=mesh,
        out_type=[
            jax.ShapeDtypeStruct((N, DIM), jnp.float32),
            jax.ShapeDtypeStruct((N, DIM), jnp.float32),
        ],
        scratch_types=[
            pltpu.VMEM((ROWS_PER_W, IDX_CHUNK), jnp.int32),
            pltpu.VMEM((2, GROUP_ROWS, DIM), jnp.float32),
            pltpu.VMEM((2, GROUP_ROWS, DIM), jnp.float32),
            pltpu.SemaphoreType.DMA,
            pltpu.SemaphoreType.DMA,
        ],
        compiler_params=pltpu.CompilerParams(use_tc_tiling_on_sc=False),
    )
    def gather2(idx_hbm, word_hbm, ctx_hbm, out_w, out_c,
                idx_v, buf_w, buf_c, sem_g, sem_o):
        wid = lax.axis_index("s") * NC + lax.axis_index("c")
        row0 = wid * ROWS_PER_W

        pltpu.sync_copy(idx_hbm.at[pl.ds(row0, ROWS_PER_W)], idx_v)

        def fire_gathers(grp, slot):
            for j in range(K):
                r = grp * K + j
                d = pl.ds(j * IDX_CHUNK, IDX_CHUNK)
                pltpu.async_copy(word_hbm.at[idx_v.at[r]],
                                 buf_w.at[slot].at[d], sem_g)
                pltpu.async_copy(ctx_hbm.at[idx_v.at[r]],
                                 buf_c.at[slot].at[d], sem_g)

        def drain_gathers():
            for j in range(K):
                d = pl.ds(j * IDX_CHUNK, IDX_CHUNK)
                pltpu.make_async_copy(word_hbm.at[idx_v.at[0]],
                                      buf_w.at[0].at[d], sem_g).wait()
                pltpu.make_async_copy(ctx_hbm.at[idx_v.at[0]],
                                      buf_c.at[0].at[d], sem_g).wait()

        def fire_writes(grp, slot):
            base = (row0 + grp * K) * IDX_CHUNK
            d = pl.ds(base, GROUP_ROWS)
            pltpu.async_copy(buf_w.at[slot], out_w.at[d], sem_o)
            pltpu.async_copy(buf_c.at[slot], out_c.at[d], sem_o)

        def drain_writes():
            d = pl.ds(0, GROUP_ROWS)
            pltpu.make_async_copy(buf_w.at[0], out_w.at[d], sem_o).wait()
            pltpu.make_async_copy(buf_c.at[0], out_c.at[d], sem_o).wait()

        fire_gathers(0, 0)

        def body(g, carry):
            slot = lax.rem(g, 2)
            nslot = lax.rem(g + 1, 2)

            @pl.when(g >= 1)
            def _():
                drain_writes()

            @pl.when(g + 1 < GROUPS)
            def _():
                fire_gathers(g + 1, nslot)

            drain_gathers()
            fire_writes(g, slot)
            return carry

        lax.fori_loop(0, GROUPS, body, 0)
        drain_writes()

    return gather2


_gather2 = _make_gather()


@jax.jit
def kernel(text, W_word, W_ctx):
    idx = text.reshape(ROWS, IDX_CHUNK).astype(jnp.int32)
    out_w, out_c = _gather2(idx, W_word, W_ctx)
    contextMatrix = out_c.reshape(BATCH, HIST, DIM)
    wordMatrix = out_w.reshape(BATCH, HIST, DIM)
    return (contextMatrix, wordMatrix)


# speedup vs baseline: 1.8941x; 1.6118x over previous
import functools
import jax
import jax.numpy as jnp
from jax import lax
from jax.experimental import pallas as pl
from jax.experimental.pallas import tpu as pltpu
from jax.experimental.pallas import tpu_sc as plsc

VOCAB = 1000000
DIM = 32
BATCH = 16384
HIST = 50

NC = 2
NS = 16
NW = NC * NS

B_PER_W = BATCH // NW
GB = 8
GROUPS = B_PER_W // GB


def _make_gather():
    mesh = plsc.VectorSubcoreMesh(core_axis_name="c", subcore_axis_name="s")

    @functools.partial(
        pl.kernel,
        mesh=mesh,
        out_type=[
            jax.ShapeDtypeStruct((BATCH, HIST, DIM), jnp.float32),
            jax.ShapeDtypeStruct((BATCH, HIST, DIM), jnp.float32),
        ],
        scratch_types=[
            pltpu.VMEM((B_PER_W, HIST), jnp.int32),
            pltpu.VMEM((2, GB, HIST, DIM), jnp.float32),
            pltpu.VMEM((2, GB, HIST, DIM), jnp.float32),
            pltpu.SemaphoreType.DMA,
            pltpu.SemaphoreType.DMA,
        ],
        compiler_params=pltpu.CompilerParams(use_tc_tiling_on_sc=False),
    )
    def gather2(idx_hbm, word_hbm, ctx_hbm, out_w, out_c,
                idx_v, buf_w, buf_c, sem_g, sem_o):
        wid = lax.axis_index("s") * NC + lax.axis_index("c")
        b0 = wid * B_PER_W

        pltpu.sync_copy(idx_hbm.at[pl.ds(b0, B_PER_W)], idx_v)

        def fire_gathers(grp, slot):
            for j in range(GB):
                b = grp * GB + j
                pltpu.async_copy(word_hbm.at[idx_v.at[b]],
                                 buf_w.at[slot].at[j], sem_g)
                pltpu.async_copy(ctx_hbm.at[idx_v.at[b]],
                                 buf_c.at[slot].at[j], sem_g)

        def drain_gathers():
            for j in range(GB):
                pltpu.make_async_copy(word_hbm.at[idx_v.at[0]],
                                      buf_w.at[0].at[j], sem_g).wait()
                pltpu.make_async_copy(ctx_hbm.at[idx_v.at[0]],
                                      buf_c.at[0].at[j], sem_g).wait()

        def fire_writes(grp, slot):
            d = pl.ds(b0 + grp * GB, GB)
            pltpu.async_copy(buf_w.at[slot], out_w.at[d], sem_o)
            pltpu.async_copy(buf_c.at[slot], out_c.at[d], sem_o)

        def drain_writes():
            d = pl.ds(0, GB)
            pltpu.make_async_copy(buf_w.at[0], out_w.at[d], sem_o).wait()
            pltpu.make_async_copy(buf_c.at[0], out_c.at[d], sem_o).wait()

        fire_gathers(0, 0)

        def body(g, carry):
            slot = lax.rem(g, 2)
            nslot = lax.rem(g + 1, 2)

            @pl.when(g >= 1)
            def _():
                drain_writes()

            @pl.when(g + 1 < GROUPS)
            def _():
                fire_gathers(g + 1, nslot)

            drain_gathers()
            fire_writes(g, slot)
            return carry

        lax.fori_loop(0, GROUPS, body, 0)
        drain_writes()

    return gather2


_gather2 = _make_gather()


@jax.jit
def kernel(text, W_word, W_ctx):
    out_w, out_c = _gather2(text.astype(jnp.int32), W_word, W_ctx)
    return (out_c, out_w)


# speedup vs baseline: 1.9458x; 1.0273x over previous
import functools
import jax
import jax.numpy as jnp
from jax import lax
from jax.experimental import pallas as pl
from jax.experimental.pallas import tpu as pltpu
from jax.experimental.pallas import tpu_sc as plsc

VOCAB = 1000000
DIM = 32
BATCH = 16384
HIST = 50

NC = 2
NS = 16
NW = NC * NS

B_PER_W = BATCH // NW
GB = 8
GROUPS = B_PER_W // GB


def _make_gather():
    mesh = plsc.VectorSubcoreMesh(core_axis_name="c", subcore_axis_name="s")

    @functools.partial(
        pl.kernel,
        mesh=mesh,
        out_type=jax.ShapeDtypeStruct((BATCH, HIST, DIM), jnp.float32),
        scratch_types=[
            pltpu.VMEM((B_PER_W, HIST), jnp.int32),
            pltpu.VMEM((2, GB, HIST, DIM), jnp.float32),
            pltpu.SemaphoreType.DMA,
            pltpu.SemaphoreType.DMA,
        ],
        compiler_params=pltpu.CompilerParams(use_tc_tiling_on_sc=False),
    )
    def gather1(idx_hbm, tab_hbm, out, idx_v, buf, sem_g, sem_o):
        wid = lax.axis_index("s") * NC + lax.axis_index("c")
        b0 = wid * B_PER_W

        pltpu.sync_copy(idx_hbm.at[pl.ds(b0, B_PER_W)], idx_v)

        def fire_gathers(grp, slot):
            for j in range(GB):
                b = grp * GB + j
                pltpu.async_copy(tab_hbm.at[idx_v.at[b]],
                                 buf.at[slot].at[j], sem_g)

        def drain_gathers():
            for j in range(GB):
                pltpu.make_async_copy(tab_hbm.at[idx_v.at[0]],
                                      buf.at[0].at[j], sem_g).wait()

        def fire_writes(grp, slot):
            d = pl.ds(b0 + grp * GB, GB)
            pltpu.async_copy(buf.at[slot], out.at[d], sem_o)

        def drain_writes():
            pltpu.make_async_copy(buf.at[0], out.at[pl.ds(0, GB)],
                                  sem_o).wait()

        fire_gathers(0, 0)

        def body(g, carry):
            slot = lax.rem(g, 2)
            nslot = lax.rem(g + 1, 2)

            @pl.when(g >= 1)
            def _():
                drain_writes()

            @pl.when(g + 1 < GROUPS)
            def _():
                fire_gathers(g + 1, nslot)

            drain_gathers()
            fire_writes(g, slot)
            return carry

        lax.fori_loop(0, GROUPS, body, 0)
        drain_writes()

    return gather1


_gather1 = _make_gather()


@jax.jit
def kernel(text, W_word, W_ctx):
    idx = text.astype(jnp.int32)
    contextMatrix = _gather1(idx, W_ctx)
    wordMatrix = _gather1(idx, W_word)
    return (contextMatrix, wordMatrix)
